# baseline (speedup 1.0000x reference)
import functools

import jax
import jax.numpy as jnp
from jax import lax
from jax.experimental import pallas as pl
from jax.experimental.pallas import tpu as pltpu
from jax.experimental.pallas import tpu_sc as plsc

DIM = 768
INTER = 1024
NUM_EXPERTS = 64
N_TOKENS = 2048
CHUNK = 128
H_SORTED = N_TOKENS + 8 * NUM_EXPERTS + CHUNK


def _router_body(x_ref, gw_ref, dest_ref, offs_ref, cnts_ref):
    xv = x_ref[:, :]
    gw = gw_ref[:, :]
    logits = lax.dot_general(xv, gw, (((1,), (1,)), ((), ())),
                             preferred_element_type=jnp.float32)
    maxv = jnp.max(logits, axis=1, keepdims=True)
    col = lax.broadcasted_iota(jnp.int32, (N_TOKENS, NUM_EXPERTS), 1)
    cand = jnp.where(logits == maxv, col, NUM_EXPERTS)
    eid = jnp.min(cand, axis=1, keepdims=True)
    onehot = (col == eid).astype(jnp.float32)

    counts = jnp.sum(onehot, axis=0, keepdims=True)
    counts8 = jnp.floor((counts + 7.0) * 0.125) * 8.0
    er = lax.broadcasted_iota(jnp.int32, (NUM_EXPERTS, NUM_EXPERTS), 0)
    ec = lax.broadcasted_iota(jnp.int32, (NUM_EXPERTS, NUM_EXPERTS), 1)
    mex = (er < ec).astype(jnp.float32)
    offs8 = lax.dot_general(counts8, mex, (((1,), (0,)), ((), ())),
                            preferred_element_type=jnp.float32)

    ri = lax.broadcasted_iota(jnp.int32, (CHUNK, CHUNK), 0)
    ci = lax.broadcasted_iota(jnp.int32, (CHUNK, CHUNK), 1)
    tril = (ci < ri).astype(jnp.float32)
    run = jnp.zeros((1, NUM_EXPERTS), jnp.float32)
    ranks_parts = []
    for c in range(N_TOKENS // CHUNK):
        oh = onehot[c * CHUNK:(c + 1) * CHUNK, :]
        rk = lax.dot_general(tril, oh, (((1,), (0,)), ((), ())),
                             preferred_element_type=jnp.float32) + run
        ranks_parts.append(rk)
        run = run + jnp.sum(oh, axis=0, keepdims=True)
    ranks = jnp.concatenate(ranks_parts, axis=0)

    dest = jnp.sum((ranks + offs8) * onehot, axis=1, keepdims=True)
    dest_ref[:, :] = dest.astype(jnp.int32)
    offs_ref[:, :] = offs8.astype(jnp.int32)
    cnts_ref[:, :] = counts.astype(jnp.int32)


def _router(x, gate_w):
    return pl.pallas_call(
        _router_body,
        out_shape=(
            jax.ShapeDtypeStruct((N_TOKENS, 1), jnp.int32),
            jax.ShapeDtypeStruct((1, NUM_EXPERTS), jnp.int32),
            jax.ShapeDtypeStruct((1, NUM_EXPERTS), jnp.int32),
        ),
    )(x, gate_w)


_NC, _NS = 2, 16
_NW = _NC * _NS
_TOK_PER_W = N_TOKENS // _NW

@functools.cache
def _sc_kernels():
    mesh = plsc.VectorSubcoreMesh(
        core_axis_name="c", subcore_axis_name="s",
        num_cores=_NC, num_subcores=_NS)
    scratch = [
        pltpu.VMEM((_TOK_PER_W,), jnp.int32),
        pltpu.VMEM((_TOK_PER_W, DIM), jnp.float32),
        pltpu.SemaphoreType.DMA,
    ]

    @functools.partial(
        pl.kernel,
        mesh=mesh,
        out_type=jax.ShapeDtypeStruct((H_SORTED, DIM), jnp.float32),
        scratch_types=scratch,
    )
    def sc_scatter(x_hbm, dest_hbm, xs_hbm, idx_v, rows_v, sem):
        wid = lax.axis_index("s") * _NC + lax.axis_index("c")
        base = wid * _TOK_PER_W
        pltpu.sync_copy(dest_hbm.at[pl.ds(base, _TOK_PER_W)], idx_v)
        pltpu.sync_copy(x_hbm.at[pl.ds(base, _TOK_PER_W)], rows_v)
        pltpu.async_copy(rows_v, xs_hbm.at[idx_v], sem).wait()

    @functools.partial(
        pl.kernel,
        mesh=mesh,
        out_type=jax.ShapeDtypeStruct((N_TOKENS, DIM), jnp.float32),
        scratch_types=scratch,
    )
    def sc_gather(os_hbm, dest_hbm, out_hbm, idx_v, rows_v, sem):
        wid = lax.axis_index("s") * _NC + lax.axis_index("c")
        base = wid * _TOK_PER_W
        pltpu.sync_copy(dest_hbm.at[pl.ds(base, _TOK_PER_W)], idx_v)
        pltpu.async_copy(os_hbm.at[idx_v], rows_v, sem).wait()
        pltpu.sync_copy(rows_v, out_hbm.at[pl.ds(base, _TOK_PER_W)])

    return sc_scatter, sc_gather


KS = 2
INTER_K = INTER // KS


def _mlp_body(offs_ref, cnts_ref, xs_ref, *w_and_out):
    wrefs, out_ref = w_and_out[:-1], w_and_out[-1]
    wg_refs = wrefs[0:KS]
    wu_refs = wrefs[KS:2 * KS]
    wd_refs = wrefs[2 * KS:3 * KS]
    e = pl.program_id(0)
    start = offs_ref[e]
    n = cnts_ref[e]
    for c in range(N_TOKENS // CHUNK):
        @pl.when(c * CHUNK < n)
        def _():
            off = pl.multiple_of(start + c * CHUNK, 8)
            xb = xs_ref[pl.ds(off, CHUNK), :]
            f = jnp.zeros((CHUNK, DIM), jnp.float32)
            for i in range(KS):
                g = lax.dot_general(xb, wg_refs[i][0], (((1,), (1,)), ((), ())),
                                    preferred_element_type=jnp.float32)
                u = lax.dot_general(xb, wu_refs[i][0], (((1,), (1,)), ((), ())),
                                    preferred_element_type=jnp.float32)
                h = (0.5 * g) * (1.0 + jnp.tanh(0.5 * g)) * u
                f = f + lax.dot_general(h, wd_refs[i][0], (((1,), (1,)), ((), ())),
                                        preferred_element_type=jnp.float32)
            out_ref[pl.ds(off, CHUNK), :] = f


def _grouped_mlp(xs, wg, wu, wd, offs, cnts):
    wg_specs = [pl.BlockSpec((1, INTER_K, DIM), lambda e, i=i: (e, i, 0))
                for i in range(KS)]
    wu_specs = [pl.BlockSpec((1, INTER_K, DIM), lambda e, i=i: (e, i, 0))
                for i in range(KS)]
    wd_specs = [pl.BlockSpec((1, DIM, INTER_K), lambda e, i=i: (e, 0, i))
                for i in range(KS)]
    return pl.pallas_call(
        _mlp_body,
        grid=(NUM_EXPERTS,),
        in_specs=[
            pl.BlockSpec(memory_space=pltpu.SMEM),
            pl.BlockSpec(memory_space=pltpu.SMEM),
            pl.BlockSpec((H_SORTED, DIM), lambda e: (0, 0)),
            *wg_specs, *wu_specs, *wd_specs,
        ],
        out_specs=pl.BlockSpec((H_SORTED, DIM), lambda e: (0, 0)),
        out_shape=jax.ShapeDtypeStruct((H_SORTED, DIM), jnp.float32),
        compiler_params=pltpu.CompilerParams(
            dimension_semantics=("arbitrary",),
            vmem_limit_bytes=120 * 1024 * 1024,
        ),
    )(offs, cnts, xs, *([wg] * KS), *([wu] * KS), *([wd] * KS))


def kernel(x, gate_w, wg, wu, wd):
    dest2d, offs2d, cnts2d = _router(x, gate_w)
    dest = dest2d.reshape(N_TOKENS)
    offs = offs2d.reshape(NUM_EXPERTS)
    cnts = cnts2d.reshape(NUM_EXPERTS)
    sc_scatter, sc_gather = _sc_kernels()
    xs = sc_scatter(x, dest)
    outs = _grouped_mlp(xs, wg, wu, wd, offs, cnts)
    return sc_gather(outs, dest)


# speedup vs baseline: 1.0223x; 1.0223x over previous
import functools

import jax
import jax.numpy as jnp
from jax import lax
from jax.experimental import pallas as pl
from jax.experimental.pallas import tpu as pltpu
from jax.experimental.pallas import tpu_sc as plsc

DIM = 768
INTER = 1024
NUM_EXPERTS = 64
N_TOKENS = 2048
CHUNK = 128
H_SORTED = N_TOKENS + 8 * NUM_EXPERTS + CHUNK


def _router_body(x_ref, gw_ref, dest_ref, offs_ref, cnts_ref):
    xv = x_ref[:, :]
    gw = gw_ref[:, :]
    logits = lax.dot_general(xv, gw, (((1,), (1,)), ((), ())),
                             preferred_element_type=jnp.float32)
    maxv = jnp.max(logits, axis=1, keepdims=True)
    col = lax.broadcasted_iota(jnp.int32, (N_TOKENS, NUM_EXPERTS), 1)
    cand = jnp.where(logits == maxv, col, NUM_EXPERTS)
    eid = jnp.min(cand, axis=1, keepdims=True)
    onehot = (col == eid).astype(jnp.float32)

    counts = jnp.sum(onehot, axis=0, keepdims=True)
    counts8 = jnp.floor((counts + 7.0) * 0.125) * 8.0
    er = lax.broadcasted_iota(jnp.int32, (NUM_EXPERTS, NUM_EXPERTS), 0)
    ec = lax.broadcasted_iota(jnp.int32, (NUM_EXPERTS, NUM_EXPERTS), 1)
    mex = (er < ec).astype(jnp.float32)
    offs8 = lax.dot_general(counts8, mex, (((1,), (0,)), ((), ())),
                            preferred_element_type=jnp.float32)

    ri = lax.broadcasted_iota(jnp.int32, (CHUNK, CHUNK), 0)
    ci = lax.broadcasted_iota(jnp.int32, (CHUNK, CHUNK), 1)
    tril = (ci < ri).astype(jnp.float32)
    run = jnp.zeros((1, NUM_EXPERTS), jnp.float32)
    ranks_parts = []
    for c in range(N_TOKENS // CHUNK):
        oh = onehot[c * CHUNK:(c + 1) * CHUNK, :]
        rk = lax.dot_general(tril, oh, (((1,), (0,)), ((), ())),
                             preferred_element_type=jnp.float32) + run
        ranks_parts.append(rk)
        run = run + jnp.sum(oh, axis=0, keepdims=True)
    ranks = jnp.concatenate(ranks_parts, axis=0)

    dest = jnp.sum((ranks + offs8) * onehot, axis=1, keepdims=True)
    dest_ref[:, :] = dest.astype(jnp.int32)
    offs_ref[:, :] = offs8.astype(jnp.int32)
    cnts_ref[:, :] = counts.astype(jnp.int32)


def _router(x, gate_w):
    return pl.pallas_call(
        _router_body,
        out_shape=(
            jax.ShapeDtypeStruct((N_TOKENS, 1), jnp.int32),
            jax.ShapeDtypeStruct((1, NUM_EXPERTS), jnp.int32),
            jax.ShapeDtypeStruct((1, NUM_EXPERTS), jnp.int32),
        ),
    )(x, gate_w)


_NC, _NS = 2, 16
_NW = _NC * _NS
_TOK_PER_W = N_TOKENS // _NW

@functools.cache
def _sc_kernels():
    mesh = plsc.VectorSubcoreMesh(
        core_axis_name="c", subcore_axis_name="s",
        num_cores=_NC, num_subcores=_NS)
    scratch = [
        pltpu.VMEM((_TOK_PER_W,), jnp.int32),
        pltpu.VMEM((_TOK_PER_W, DIM), jnp.float32),
        pltpu.SemaphoreType.DMA,
    ]

    @functools.partial(
        pl.kernel,
        mesh=mesh,
        out_type=jax.ShapeDtypeStruct((H_SORTED, DIM), jnp.float32),
        scratch_types=scratch,
    )
    def sc_scatter(x_hbm, dest_hbm, xs_hbm, idx_v, rows_v, sem):
        wid = lax.axis_index("s") * _NC + lax.axis_index("c")
        base = wid * _TOK_PER_W
        pltpu.sync_copy(dest_hbm.at[pl.ds(base, _TOK_PER_W)], idx_v)
        pltpu.sync_copy(x_hbm.at[pl.ds(base, _TOK_PER_W)], rows_v)
        pltpu.async_copy(rows_v, xs_hbm.at[idx_v], sem).wait()

    @functools.partial(
        pl.kernel,
        mesh=mesh,
        out_type=jax.ShapeDtypeStruct((N_TOKENS, DIM), jnp.float32),
        scratch_types=scratch,
    )
    def sc_gather(os_hbm, dest_hbm, out_hbm, idx_v, rows_v, sem):
        wid = lax.axis_index("s") * _NC + lax.axis_index("c")
        base = wid * _TOK_PER_W
        pltpu.sync_copy(dest_hbm.at[pl.ds(base, _TOK_PER_W)], idx_v)
        pltpu.async_copy(os_hbm.at[idx_v], rows_v, sem).wait()
        pltpu.sync_copy(rows_v, out_hbm.at[pl.ds(base, _TOK_PER_W)])

    return sc_scatter, sc_gather


KS = 2
INTER_K = INTER // KS


def _mlp_body(offs_ref, cnts_ref, xs_ref, *w_and_out):
    wrefs, out_ref = w_and_out[:-1], w_and_out[-1]
    wg_refs = wrefs[0:KS]
    wu_refs = wrefs[KS:2 * KS]
    wd_refs = wrefs[2 * KS:3 * KS]
    e = pl.program_id(0)
    start = offs_ref[e]
    n = cnts_ref[e]
    for c in range(N_TOKENS // CHUNK):
        @pl.when(c * CHUNK < n)
        def _():
            off = pl.multiple_of(start + c * CHUNK, 8)
            xb = xs_ref[pl.ds(off, CHUNK), :]
            hs = []
            for i in range(KS):
                g = lax.dot_general(xb, wg_refs[i][0], (((1,), (1,)), ((), ())),
                                    preferred_element_type=jnp.float32)
                u = lax.dot_general(xb, wu_refs[i][0], (((1,), (1,)), ((), ())),
                                    preferred_element_type=jnp.float32)
                hs.append((0.5 * g) * (1.0 + jnp.tanh(0.5 * g)) * u)
            h = jnp.concatenate(hs, axis=1)
            for i in range(KS):
                f = lax.dot_general(h, wd_refs[i][0], (((1,), (1,)), ((), ())),
                                    preferred_element_type=jnp.float32)
                out_ref[pl.ds(off, CHUNK), (DIM // KS) * i:(DIM // KS) * (i + 1)] = f


def _grouped_mlp(xs, wg, wu, wd, offs, cnts):
    wg_specs = [pl.BlockSpec((1, INTER_K, DIM), lambda e, i=i: (e, i, 0))
                for i in range(KS)]
    wu_specs = [pl.BlockSpec((1, INTER_K, DIM), lambda e, i=i: (e, i, 0))
                for i in range(KS)]
    wd_specs = [pl.BlockSpec((1, DIM // KS, INTER), lambda e, i=i: (e, i, 0))
                for i in range(KS)]
    return pl.pallas_call(
        _mlp_body,
        grid=(NUM_EXPERTS,),
        in_specs=[
            pl.BlockSpec(memory_space=pltpu.SMEM),
            pl.BlockSpec(memory_space=pltpu.SMEM),
            pl.BlockSpec((H_SORTED, DIM), lambda e: (0, 0)),
            *wg_specs, *wu_specs, *wd_specs,
        ],
        out_specs=pl.BlockSpec((H_SORTED, DIM), lambda e: (0, 0)),
        out_shape=jax.ShapeDtypeStruct((H_SORTED, DIM), jnp.float32),
        compiler_params=pltpu.CompilerParams(
            dimension_semantics=("arbitrary",),
            vmem_limit_bytes=120 * 1024 * 1024,
        ),
    )(offs, cnts, xs, *([wg] * KS), *([wu] * KS), *([wd] * KS))


def kernel(x, gate_w, wg, wu, wd):
    dest2d, offs2d, cnts2d = _router(x, gate_w)
    dest = dest2d.reshape(N_TOKENS)
    offs = offs2d.reshape(NUM_EXPERTS)
    cnts = cnts2d.reshape(NUM_EXPERTS)
    sc_scatter, sc_gather = _sc_kernels()
    xs = sc_scatter(x, dest)
    outs = _grouped_mlp(xs, wg, wu, wd, offs, cnts)
    return sc_gather(outs, dest)


# speedup vs baseline: 1.0355x; 1.0129x over previous
import functools

import jax
import jax.numpy as jnp
from jax import lax
from jax.experimental import pallas as pl
from jax.experimental.pallas import tpu as pltpu
from jax.experimental.pallas import tpu_sc as plsc

DIM = 768
INTER = 1024
NUM_EXPERTS = 64
N_TOKENS = 2048
CHUNK = 128
H_SORTED = N_TOKENS + 8 * NUM_EXPERTS + CHUNK


def _router_body(x_ref, gw_ref, dest_ref, offs_ref, cnts_ref):
    xv = x_ref[:, :]
    gw = gw_ref[:, :]
    logits = lax.dot_general(xv, gw, (((1,), (1,)), ((), ())),
                             preferred_element_type=jnp.float32)
    maxv = jnp.max(logits, axis=1, keepdims=True)
    col = lax.broadcasted_iota(jnp.int32, (N_TOKENS, NUM_EXPERTS), 1)
    cand = jnp.where(logits == maxv, col, NUM_EXPERTS)
    eid = jnp.min(cand, axis=1, keepdims=True)
    onehot = (col == eid).astype(jnp.float32)

    counts = jnp.sum(onehot, axis=0, keepdims=True)
    counts8 = jnp.floor((counts + 7.0) * 0.125) * 8.0
    er = lax.broadcasted_iota(jnp.int32, (NUM_EXPERTS, NUM_EXPERTS), 0)
    ec = lax.broadcasted_iota(jnp.int32, (NUM_EXPERTS, NUM_EXPERTS), 1)
    mex = (er < ec).astype(jnp.float32)
    offs8 = lax.dot_general(counts8, mex, (((1,), (0,)), ((), ())),
                            preferred_element_type=jnp.float32)

    ri = lax.broadcasted_iota(jnp.int32, (CHUNK, CHUNK), 0)
    ci = lax.broadcasted_iota(jnp.int32, (CHUNK, CHUNK), 1)
    tril = (ci < ri).astype(jnp.float32)
    run = jnp.zeros((1, NUM_EXPERTS), jnp.float32)
    ranks_parts = []
    for c in range(N_TOKENS // CHUNK):
        oh = onehot[c * CHUNK:(c + 1) * CHUNK, :]
        rk = lax.dot_general(tril, oh, (((1,), (0,)), ((), ())),
                             preferred_element_type=jnp.float32) + run
        ranks_parts.append(rk)
        run = run + jnp.sum(oh, axis=0, keepdims=True)
    ranks = jnp.concatenate(ranks_parts, axis=0)

    dest = jnp.sum((ranks + offs8) * onehot, axis=1, keepdims=True)
    dest_ref[:, :] = dest.astype(jnp.int32)
    offs_ref[:, :] = offs8.astype(jnp.int32)
    cnts_ref[:, :] = counts.astype(jnp.int32)


def _router(x, gate_w):
    return pl.pallas_call(
        _router_body,
        out_shape=(
            jax.ShapeDtypeStruct((N_TOKENS, 1), jnp.int32),
            jax.ShapeDtypeStruct((1, NUM_EXPERTS), jnp.int32),
            jax.ShapeDtypeStruct((1, NUM_EXPERTS), jnp.int32),
        ),
    )(x, gate_w)


_NC, _NS = 2, 16
_NW = _NC * _NS
_TOK_PER_W = N_TOKENS // _NW

@functools.cache
def _sc_kernels():
    mesh = plsc.VectorSubcoreMesh(
        core_axis_name="c", subcore_axis_name="s",
        num_cores=_NC, num_subcores=_NS)
    scratch = [
        pltpu.VMEM((_TOK_PER_W,), jnp.int32),
        pltpu.VMEM((_TOK_PER_W, DIM), jnp.float32),
        pltpu.SemaphoreType.DMA,
    ]

    @functools.partial(
        pl.kernel,
        mesh=mesh,
        out_type=jax.ShapeDtypeStruct((H_SORTED, DIM), jnp.float32),
        scratch_types=scratch,
    )
    def sc_scatter(x_hbm, dest_hbm, xs_hbm, idx_v, rows_v, sem):
        wid = lax.axis_index("s") * _NC + lax.axis_index("c")
        base = wid * _TOK_PER_W
        pltpu.sync_copy(dest_hbm.at[pl.ds(base, _TOK_PER_W)], idx_v)
        pltpu.sync_copy(x_hbm.at[pl.ds(base, _TOK_PER_W)], rows_v)
        pltpu.async_copy(rows_v, xs_hbm.at[idx_v], sem).wait()

    @functools.partial(
        pl.kernel,
        mesh=mesh,
        out_type=jax.ShapeDtypeStruct((N_TOKENS, DIM), jnp.float32),
        scratch_types=scratch,
    )
    def sc_gather(os_hbm, dest_hbm, out_hbm, idx_v, rows_v, sem):
        wid = lax.axis_index("s") * _NC + lax.axis_index("c")
        base = wid * _TOK_PER_W
        pltpu.sync_copy(dest_hbm.at[pl.ds(base, _TOK_PER_W)], idx_v)
        pltpu.async_copy(os_hbm.at[idx_v], rows_v, sem).wait()
        pltpu.sync_copy(rows_v, out_hbm.at[pl.ds(base, _TOK_PER_W)])

    return sc_scatter, sc_gather


KSU = 4
KSD = 2
INTER_K = INTER // KSU


def _mlp_body(offs_ref, cnts_ref, xs_ref, *w_and_out):
    wrefs, out_ref = w_and_out[:-1], w_and_out[-1]
    wg_refs = wrefs[0:KSU]
    wu_refs = wrefs[KSU:2 * KSU]
    wd_refs = wrefs[2 * KSU:2 * KSU + KSD]
    e = pl.program_id(0)
    start = offs_ref[e]
    n = cnts_ref[e]
    for c in range(N_TOKENS // CHUNK):
        @pl.when(c * CHUNK < n)
        def _():
            off = pl.multiple_of(start + c * CHUNK, 8)
            xb = xs_ref[pl.ds(off, CHUNK), :]
            hs = []
            for i in range(KSU):
                g = lax.dot_general(xb, wg_refs[i][0], (((1,), (1,)), ((), ())),
                                    preferred_element_type=jnp.float32)
                u = lax.dot_general(xb, wu_refs[i][0], (((1,), (1,)), ((), ())),
                                    preferred_element_type=jnp.float32)
                hs.append((0.5 * g) * (1.0 + jnp.tanh(0.5 * g)) * u)
            h = jnp.concatenate(hs, axis=1)
            for i in range(KSD):
                f = lax.dot_general(h, wd_refs[i][0], (((1,), (1,)), ((), ())),
                                    preferred_element_type=jnp.float32)
                out_ref[pl.ds(off, CHUNK), (DIM // KSD) * i:(DIM // KSD) * (i + 1)] = f


def _grouped_mlp(xs, wg, wu, wd, offs, cnts):
    wg_specs = [pl.BlockSpec((1, INTER_K, DIM), lambda e, i=i: (e, i, 0))
                for i in range(KSU)]
    wu_specs = [pl.BlockSpec((1, INTER_K, DIM), lambda e, i=i: (e, i, 0))
                for i in range(KSU)]
    wd_specs = [pl.BlockSpec((1, DIM // KSD, INTER), lambda e, i=i: (e, i, 0))
                for i in range(KSD)]
    return pl.pallas_call(
        _mlp_body,
        grid=(NUM_EXPERTS,),
        in_specs=[
            pl.BlockSpec(memory_space=pltpu.SMEM),
            pl.BlockSpec(memory_space=pltpu.SMEM),
            pl.BlockSpec((H_SORTED, DIM), lambda e: (0, 0)),
            *wg_specs, *wu_specs, *wd_specs,
        ],
        out_specs=pl.BlockSpec((H_SORTED, DIM), lambda e: (0, 0)),
        out_shape=jax.ShapeDtypeStruct((H_SORTED, DIM), jnp.float32),
        compiler_params=pltpu.CompilerParams(
            dimension_semantics=("arbitrary",),
            vmem_limit_bytes=120 * 1024 * 1024,
        ),
    )(offs, cnts, xs, *([wg] * KSU), *([wu] * KSU), *([wd] * KSD))


def kernel(x, gate_w, wg, wu, wd):
    dest2d, offs2d, cnts2d = _router(x, gate_w)
    dest = dest2d.reshape(N_TOKENS)
    offs = offs2d.reshape(NUM_EXPERTS)
    cnts = cnts2d.reshape(NUM_EXPERTS)
    sc_scatter, sc_gather = _sc_kernels()
    xs = sc_scatter(x, dest)
    outs = _grouped_mlp(xs, wg, wu, wd, offs, cnts)
    return sc_gather(outs, dest)


# speedup vs baseline: 1.0621x; 1.0257x over previous
import functools

import jax
import jax.numpy as jnp
from jax import lax
from jax.experimental import pallas as pl
from jax.experimental.pallas import tpu as pltpu
from jax.experimental.pallas import tpu_sc as plsc

DIM = 768
INTER = 1024
NUM_EXPERTS = 64
N_TOKENS = 2048
CHUNK = 128
H_SORTED = N_TOKENS + 8 * NUM_EXPERTS + CHUNK


def _router_body(x_ref, gw_ref, dest_ref, offs_ref, cnts_ref):
    xv = x_ref[:, :]
    gw = gw_ref[:, :]
    logits = lax.dot_general(xv, gw, (((1,), (1,)), ((), ())),
                             preferred_element_type=jnp.float32)
    maxv = jnp.max(logits, axis=1, keepdims=True)
    col = lax.broadcasted_iota(jnp.int32, (N_TOKENS, NUM_EXPERTS), 1)
    cand = jnp.where(logits == maxv, col, NUM_EXPERTS)
    eid = jnp.min(cand, axis=1, keepdims=True)
    onehot = (col == eid).astype(jnp.float32)

    counts = jnp.sum(onehot, axis=0, keepdims=True)
    counts8 = jnp.floor((counts + 7.0) * 0.125) * 8.0
    er = lax.broadcasted_iota(jnp.int32, (NUM_EXPERTS, NUM_EXPERTS), 0)
    ec = lax.broadcasted_iota(jnp.int32, (NUM_EXPERTS, NUM_EXPERTS), 1)
    mex = (er < ec).astype(jnp.float32)
    offs8 = lax.dot_general(counts8, mex, (((1,), (0,)), ((), ())),
                            preferred_element_type=jnp.float32)

    ri = lax.broadcasted_iota(jnp.int32, (CHUNK, CHUNK), 0)
    ci = lax.broadcasted_iota(jnp.int32, (CHUNK, CHUNK), 1)
    tril = (ci < ri).astype(jnp.float32)
    run = jnp.zeros((1, NUM_EXPERTS), jnp.float32)
    ranks_parts = []
    for c in range(N_TOKENS // CHUNK):
        oh = onehot[c * CHUNK:(c + 1) * CHUNK, :]
        rk = lax.dot_general(tril, oh, (((1,), (0,)), ((), ())),
                             preferred_element_type=jnp.float32) + run
        ranks_parts.append(rk)
        run = run + jnp.sum(oh, axis=0, keepdims=True)
    ranks = jnp.concatenate(ranks_parts, axis=0)

    dest = jnp.sum((ranks + offs8) * onehot, axis=1, keepdims=True)
    dest_ref[:, :] = dest.astype(jnp.int32)
    offs_ref[:, :] = offs8.astype(jnp.int32)
    cnts_ref[:, :] = counts.astype(jnp.int32)


def _router(x, gate_w):
    return pl.pallas_call(
        _router_body,
        out_shape=(
            jax.ShapeDtypeStruct((N_TOKENS, 1), jnp.int32),
            jax.ShapeDtypeStruct((1, NUM_EXPERTS), jnp.int32),
            jax.ShapeDtypeStruct((1, NUM_EXPERTS), jnp.int32),
        ),
    )(x, gate_w)


_NC, _NS = 2, 16
_NW = _NC * _NS
_TOK_PER_W = N_TOKENS // _NW

@functools.cache
def _sc_kernels():
    mesh = plsc.VectorSubcoreMesh(
        core_axis_name="c", subcore_axis_name="s",
        num_cores=_NC, num_subcores=_NS)
    scratch = [
        pltpu.VMEM((_TOK_PER_W,), jnp.int32),
        pltpu.VMEM((_TOK_PER_W, DIM), jnp.float32),
        pltpu.SemaphoreType.DMA,
    ]

    @functools.partial(
        pl.kernel,
        mesh=mesh,
        out_type=jax.ShapeDtypeStruct((H_SORTED, DIM), jnp.float32),
        scratch_types=scratch,
    )
    def sc_scatter(x_hbm, dest_hbm, xs_hbm, idx_v, rows_v, sem):
        wid = lax.axis_index("s") * _NC + lax.axis_index("c")
        base = wid * _TOK_PER_W
        pltpu.sync_copy(dest_hbm.at[pl.ds(base, _TOK_PER_W)], idx_v)
        pltpu.sync_copy(x_hbm.at[pl.ds(base, _TOK_PER_W)], rows_v)
        pltpu.async_copy(rows_v, xs_hbm.at[idx_v], sem).wait()

    @functools.partial(
        pl.kernel,
        mesh=mesh,
        out_type=jax.ShapeDtypeStruct((N_TOKENS, DIM), jnp.float32),
        scratch_types=scratch,
    )
    def sc_gather(os_hbm, dest_hbm, out_hbm, idx_v, rows_v, sem):
        wid = lax.axis_index("s") * _NC + lax.axis_index("c")
        base = wid * _TOK_PER_W
        pltpu.sync_copy(dest_hbm.at[pl.ds(base, _TOK_PER_W)], idx_v)
        pltpu.async_copy(os_hbm.at[idx_v], rows_v, sem).wait()
        pltpu.sync_copy(rows_v, out_hbm.at[pl.ds(base, _TOK_PER_W)])

    return sc_scatter, sc_gather


KSU = 8
KSD = 2
INTER_K = INTER // KSU


def _mlp_body(offs_ref, cnts_ref, xs_ref, *w_and_out):
    wrefs, out_ref = w_and_out[:-1], w_and_out[-1]
    wg_refs = wrefs[0:KSU]
    wu_refs = wrefs[KSU:2 * KSU]
    wd_refs = wrefs[2 * KSU:2 * KSU + KSD]
    e = pl.program_id(0)
    start = offs_ref[e]
    n = cnts_ref[e]
    for c in range(N_TOKENS // CHUNK):
        @pl.when(c * CHUNK < n)
        def _():
            off = pl.multiple_of(start + c * CHUNK, 8)
            xb = xs_ref[pl.ds(off, CHUNK), :]
            hs = []
            for i in range(KSU):
                g = lax.dot_general(xb, wg_refs[i][0], (((1,), (1,)), ((), ())),
                                    preferred_element_type=jnp.float32)
                u = lax.dot_general(xb, wu_refs[i][0], (((1,), (1,)), ((), ())),
                                    preferred_element_type=jnp.float32)
                hs.append((0.5 * g) * (1.0 + jnp.tanh(0.5 * g)) * u)
            h = jnp.concatenate(hs, axis=1)
            for i in range(KSD):
                f = lax.dot_general(h, wd_refs[i][0], (((1,), (1,)), ((), ())),
                                    preferred_element_type=jnp.float32)
                out_ref[pl.ds(off, CHUNK), (DIM // KSD) * i:(DIM // KSD) * (i + 1)] = f


def _grouped_mlp(xs, wg, wu, wd, offs, cnts):
    wg_specs = [pl.BlockSpec((1, INTER_K, DIM), lambda e, i=i: (e, i, 0))
                for i in range(KSU)]
    wu_specs = [pl.BlockSpec((1, INTER_K, DIM), lambda e, i=i: (e, i, 0))
                for i in range(KSU)]
    wd_specs = [pl.BlockSpec((1, DIM // KSD, INTER), lambda e, i=i: (e, i, 0))
                for i in range(KSD)]
    return pl.pallas_call(
        _mlp_body,
        grid=(NUM_EXPERTS,),
        in_specs=[
            pl.BlockSpec(memory_space=pltpu.SMEM),
            pl.BlockSpec(memory_space=pltpu.SMEM),
            pl.BlockSpec((H_SORTED, DIM), lambda e: (0, 0)),
            *wg_specs, *wu_specs, *wd_specs,
        ],
        out_specs=pl.BlockSpec((H_SORTED, DIM), lambda e: (0, 0)),
        out_shape=jax.ShapeDtypeStruct((H_SORTED, DIM), jnp.float32),
        compiler_params=pltpu.CompilerParams(
            dimension_semantics=("arbitrary",),
            vmem_limit_bytes=120 * 1024 * 1024,
        ),
    )(offs, cnts, xs, *([wg] * KSU), *([wu] * KSU), *([wd] * KSD))


def kernel(x, gate_w, wg, wu, wd):
    dest2d, offs2d, cnts2d = _router(x, gate_w)
    dest = dest2d.reshape(N_TOKENS)
    offs = offs2d.reshape(NUM_EXPERTS)
    cnts = cnts2d.reshape(NUM_EXPERTS)
    sc_scatter, sc_gather = _sc_kernels()
    xs = sc_scatter(x, dest)
    outs = _grouped_mlp(xs, wg, wu, wd, offs, cnts)
    return sc_gather(outs, dest)
